# Initial kernel scaffold; baseline (speedup 1.0000x reference)
import jax
import jax.numpy as jnp
from jax.experimental import pallas as pl


def kernel(x, edge_index, batch, W1, b1, g1, be1, W2, b2, g2, be2, Wc, bc):
    raise NotImplementedError("write your pallas kernel here")


# speedup vs baseline: 18.1722x; 18.1722x over previous
import functools
import math

import jax
import jax.numpy as jnp
from jax import lax
from jax.experimental import pallas as pl
from jax.experimental.pallas import tpu as pltpu
from jax.experimental.pallas import tpu_sc as plsc

_N_GRAPHS = 64
_CH = 128
_DW = 8
_NC = 2
_NS = 16
_NW = _NC * _NS


def _sc_degree(dst3, ones, zeros):
    nw, k_chunks, ch = dst3.shape
    n_acc = zeros.shape[0] * _NS
    rpt = n_acc // _NS
    mesh = plsc.VectorSubcoreMesh(core_axis_name="c", subcore_axis_name="s")

    @functools.partial(
        pl.kernel,
        out_type=jax.ShapeDtypeStruct((_NC, n_acc, _DW), jnp.float32),
        mesh=mesh,
        scratch_types=[
            pltpu.VMEM((k_chunks, ch), jnp.int32),
            pltpu.VMEM((ch, _DW), jnp.float32),
            pltpu.VMEM_SHARED((n_acc, _DW), jnp.float32),
        ],
        compiler_params=pltpu.CompilerParams(use_tc_tiling_on_sc=False),
    )
    def deg_kernel(dst_hbm, ones_hbm, zeros_hbm, out_hbm, dst_v, ones_v, acc):
        cid = lax.axis_index("c")
        sid = lax.axis_index("s")
        wid = sid * _NC + cid
        pltpu.sync_copy(zeros_hbm, acc.at[pl.ds(sid * rpt, rpt)])
        pltpu.sync_copy(dst_hbm.at[wid], dst_v)
        pltpu.sync_copy(ones_hbm, ones_v)
        plsc.subcore_barrier()

        def body(j, carry):
            pltpu.sync_copy(ones_v, acc.at[dst_v.at[j]], add=True)
            return carry

        lax.fori_loop(0, k_chunks, body, 0)
        plsc.subcore_barrier()
        pltpu.sync_copy(acc.at[pl.ds(sid * rpt, rpt)],
                        out_hbm.at[cid, pl.ds(sid * rpt, rpt)])

    return deg_kernel(dst3, ones, zeros)


def _sc_conv_scatter(hp, src3, dst3, zeros):
    n_nodes, h_dim = hp.shape
    nw, k_chunks, ch = src3.shape
    n_acc = zeros.shape[0] * _NS
    rpt = n_acc // _NS
    mesh = plsc.VectorSubcoreMesh(core_axis_name="c", subcore_axis_name="s")

    @functools.partial(
        pl.kernel,
        out_type=jax.ShapeDtypeStruct((_NC, n_acc, h_dim), jnp.float32),
        mesh=mesh,
        scratch_types=[
            pltpu.VMEM((k_chunks, ch), jnp.int32),
            pltpu.VMEM((k_chunks, ch), jnp.int32),
            pltpu.VMEM((ch, h_dim), jnp.float32),
            pltpu.VMEM_SHARED((n_acc, h_dim), jnp.float32),
            pltpu.SemaphoreType.DMA,
        ],
        compiler_params=pltpu.CompilerParams(use_tc_tiling_on_sc=False),
    )
    def conv_kernel(hp_hbm, src_hbm, dst_hbm, zeros_hbm, out_hbm,
                    src_v, dst_v, rows_v, acc, sem):
        cid = lax.axis_index("c")
        sid = lax.axis_index("s")
        wid = sid * _NC + cid
        pltpu.sync_copy(zeros_hbm, acc.at[pl.ds(sid * rpt, rpt)])
        pltpu.sync_copy(src_hbm.at[wid], src_v)
        pltpu.sync_copy(dst_hbm.at[wid], dst_v)
        plsc.subcore_barrier()

        def body(j, carry):
            pltpu.async_copy(hp_hbm.at[src_v.at[j]], rows_v, sem).wait()
            pltpu.sync_copy(rows_v, acc.at[dst_v.at[j]], add=True)
            return carry

        lax.fori_loop(0, k_chunks, body, 0)
        plsc.subcore_barrier()
        pltpu.sync_copy(acc.at[pl.ds(sid * rpt, rpt)],
                        out_hbm.at[cid, pl.ds(sid * rpt, rpt)])

    return conv_kernel(hp, src3, dst3, zeros)


def _tc_in_proj(x, w1, degp):
    n, _ = x.shape
    h_dim = w1.shape[1]

    def body(x_ref, w_ref, degp_ref, hp_ref, dis_ref):
        dp = degp_ref[...]
        deg = lax.slice(dp[0] + dp[1], (0, 0), (n, 1)) + 1.0
        dis = lax.rsqrt(deg)
        h = jnp.dot(x_ref[...], w_ref[...], preferred_element_type=jnp.float32)
        hp_ref[...] = h * dis
        dis_ref[...] = dis

    return pl.pallas_call(
        body,
        out_shape=[jax.ShapeDtypeStruct((n, h_dim), jnp.float32),
                   jax.ShapeDtypeStruct((n, 1), jnp.float32)],
    )(x, w1, degp)


def _tc_mid(part, hp, dis, b1, g1, be1, w2):
    n, h_dim = hp.shape

    def body(part_ref, hp_ref, dis_ref, b_ref, g_ref, be_ref, w_ref, out_ref):
        p = part_ref[...]
        dis_v = dis_ref[...]
        psum = lax.slice(p[0] + p[1], (0, 0), (n, h_dim))
        agg = (psum + hp_ref[...]) * dis_v + b_ref[...]
        mu = jnp.mean(agg, axis=0, keepdims=True)
        var = jnp.mean((agg - mu) ** 2, axis=0, keepdims=True)
        h1 = (agg - mu) * lax.rsqrt(var + 1e-5) * g_ref[...] + be_ref[...]
        h1 = jnp.maximum(h1, 0.0)
        out_ref[...] = jnp.dot(h1, w_ref[...],
                               preferred_element_type=jnp.float32) * dis_v

    return pl.pallas_call(
        body,
        out_shape=jax.ShapeDtypeStruct((n, h_dim), jnp.float32),
    )(part, hp, dis, b1, g1, be1, w2)


def _tc_pool_head(part, hp, dis, b2, g2, be2, batch2d, wc, bc):
    n, h_dim = hp.shape
    n_graphs = _N_GRAPHS

    def body(part_ref, hp_ref, dis_ref, b_ref, g_ref, be_ref, batch_ref,
             wc_ref, bc_ref, out_ref, mx_ref):
        p = part_ref[...]
        psum = lax.slice(p[0] + p[1], (0, 0), (n, h_dim))
        agg = (psum + hp_ref[...]) * dis_ref[...] + b_ref[...]
        mu = jnp.mean(agg, axis=0, keepdims=True)
        var = jnp.mean((agg - mu) ** 2, axis=0, keepdims=True)
        h = (agg - mu) * lax.rsqrt(var + 1e-5) * g_ref[...] + be_ref[...]
        h = jnp.maximum(h, 0.0)

        gids = lax.broadcasted_iota(jnp.int32, (1, n_graphs), 1)
        batch_v = batch_ref[...]
        mask = (batch_v == gids).astype(jnp.float32)
        s = lax.dot_general(mask, h, (((0,), (0,)), ((), ())),
                            preferred_element_type=jnp.float32)
        ones_col = jnp.ones((n, 1), jnp.float32)
        cnt = lax.dot_general(mask, ones_col, (((0,), (0,)), ((), ())),
                              preferred_element_type=jnp.float32)
        mean = s / jnp.maximum(cnt, 1.0)

        def gbody(g, carry):
            col = (batch_v == g).astype(jnp.float32)
            m = jnp.max(h * col, axis=0, keepdims=True)
            mx_ref[pl.ds(g, 1), :] = m
            return carry

        lax.fori_loop(0, n_graphs, gbody, 0)

        pooled = jnp.concatenate([mean, mx_ref[...], s], axis=1)
        out_ref[...] = jnp.dot(pooled, wc_ref[...],
                               preferred_element_type=jnp.float32) + bc_ref[...]

    return pl.pallas_call(
        body,
        out_shape=jax.ShapeDtypeStruct((n_graphs, wc.shape[1]), jnp.float32),
        scratch_shapes=[pltpu.VMEM((n_graphs, h_dim), jnp.float32)],
    )(part, hp, dis, b2, g2, be2, batch2d, wc, bc)


def _pad_geometry(n_nodes, n_edges):
    k_chunks = math.ceil(n_edges / (_NW * _CH))
    pad_e = _NW * k_chunks * _CH
    n_acc = ((n_nodes + 1 + 127) // 128) * 128
    return k_chunks, pad_e, n_acc


def kernel(x, edge_index, batch, W1, b1, g1, be1, W2, b2, g2, be2, Wc, bc):
    n, _ = x.shape
    h_dim = W1.shape[1]
    n_edges = edge_index.shape[1]
    k_chunks, pad_e, n_acc = _pad_geometry(n, n_edges)

    src = jnp.concatenate(
        [edge_index[0], jnp.zeros((pad_e - n_edges,), jnp.int32)])
    dst = jnp.concatenate(
        [edge_index[1], jnp.full((pad_e - n_edges,), n, jnp.int32)])
    src3 = src.reshape(_NW, k_chunks, _CH)
    dst3 = dst.reshape(_NW, k_chunks, _CH)

    ones = jnp.ones((_CH, _DW), jnp.float32)
    zeros_deg = jnp.zeros((n_acc // _NS, _DW), jnp.float32)
    zeros_conv = jnp.zeros((n_acc // _NS, h_dim), jnp.float32)

    b1r, g1r, be1r = b1.reshape(1, -1), g1.reshape(1, -1), be1.reshape(1, -1)
    b2r, g2r, be2r = b2.reshape(1, -1), g2.reshape(1, -1), be2.reshape(1, -1)
    bcr = bc.reshape(1, -1)
    batch2d = batch.reshape(-1, 1)

    degp = _sc_degree(dst3, ones, zeros_deg)
    hp1, dis = _tc_in_proj(x, W1, degp)
    part1 = _sc_conv_scatter(hp1, src3, dst3, zeros_conv)
    hp2 = _tc_mid(part1, hp1, dis, b1r, g1r, be1r, W2)
    part2 = _sc_conv_scatter(hp2, src3, dst3, zeros_conv)
    return _tc_pool_head(part2, hp2, dis, b2r, g2r, be2r, batch2d, Wc, bcr)
